# baseline (device time: 8611 ns/iter reference)
import jax
import jax.numpy as jnp
from jax import lax
from jax.experimental import pallas as pl
from jax.experimental.pallas import tpu as pltpu

N_DEV = 8


def kernel(x):
    m_per, n_per = x.shape
    sub = m_per // 128

    def body(x_ref, out_ref, stats_ref, send_sems, recv_sems):
        my = lax.axis_index("i")

        barrier_sem = pltpu.get_barrier_semaphore()
        for k in range(N_DEV):
            @pl.when(k != my)
            def _():
                pl.semaphore_signal(
                    barrier_sem, inc=1,
                    device_id=(k,), device_id_type=pl.DeviceIdType.MESH,
                )

        xv = x_ref[:, :]
        m = jnp.max(xv, axis=1, keepdims=True)
        p = jnp.exp(xv - m)
        s = jnp.sum(p, axis=1, keepdims=True)
        m_pk = jnp.reshape(m, (sub, 128))
        s_pk = jnp.reshape(s, (sub, 128))
        mine = jnp.concatenate([m_pk, s_pk], axis=0)
        stats_ref[my] = mine

        pl.semaphore_wait(barrier_sem, N_DEV - 1)

        for k in range(N_DEV):
            @pl.when(k != my)
            def _():
                pltpu.make_async_remote_copy(
                    src_ref=stats_ref.at[my],
                    dst_ref=stats_ref.at[my],
                    send_sem=send_sems.at[k],
                    recv_sem=recv_sems.at[my],
                    device_id=(k,),
                    device_id_type=pl.DeviceIdType.MESH,
                ).start()

        for k in range(N_DEV):
            @pl.when(k != my)
            def _():
                pltpu.make_async_remote_copy(
                    src_ref=stats_ref.at[k],
                    dst_ref=stats_ref.at[k],
                    send_sem=send_sems.at[k],
                    recv_sem=recv_sems.at[k],
                    device_id=(k,),
                    device_id_type=pl.DeviceIdType.MESH,
                ).wait_recv()

        g = stats_ref[:, :, :]
        gm = g[:, 0:sub, :]
        gs = g[:, sub:, :]
        gmax = jnp.max(gm, axis=0)
        denom = jnp.sum(gs * jnp.exp(gm - gmax), axis=0)
        scale_pk = jnp.exp(m_pk - gmax) / denom
        out_ref[:, :] = p * jnp.reshape(scale_pk, (m_per, 1))

        for k in range(N_DEV):
            @pl.when(k != my)
            def _():
                pltpu.make_async_remote_copy(
                    src_ref=stats_ref.at[my],
                    dst_ref=stats_ref.at[my],
                    send_sem=send_sems.at[k],
                    recv_sem=recv_sems.at[my],
                    device_id=(k,),
                    device_id_type=pl.DeviceIdType.MESH,
                ).wait_send()

    return pl.pallas_call(
        body,
        out_shape=jax.ShapeDtypeStruct((m_per, n_per), jnp.float32),
        in_specs=[pl.BlockSpec(memory_space=pltpu.VMEM)],
        out_specs=pl.BlockSpec(memory_space=pltpu.VMEM),
        scratch_shapes=[
            pltpu.VMEM((N_DEV, 8, 128), jnp.float32),
            pltpu.SemaphoreType.DMA((N_DEV,)),
            pltpu.SemaphoreType.DMA((N_DEV,)),
        ],
        compiler_params=pltpu.CompilerParams(collective_id=0),
    )(x)


# device time: 8596 ns/iter; 1.0017x vs baseline; 1.0017x over previous
import jax
import jax.numpy as jnp
from jax import lax
from jax.experimental import pallas as pl
from jax.experimental.pallas import tpu as pltpu

N_DEV = 8


def kernel(x):
    m_per, n_per = x.shape
    sub = m_per // 128

    def body(x_ref, out_ref, stats_ref, send_sems, recv_sems):
        my = lax.axis_index("i")

        barrier_sem = pltpu.get_barrier_semaphore()
        for k in range(N_DEV):
            @pl.when(k != my)
            def _():
                pl.semaphore_signal(
                    barrier_sem, inc=1,
                    device_id=(k,), device_id_type=pl.DeviceIdType.MESH,
                )

        xv = x_ref[:, :]
        m = jnp.max(xv, axis=1, keepdims=True)
        p = jnp.exp(xv - m)
        s = jnp.sum(p, axis=1, keepdims=True)
        m_pk = jnp.reshape(m, (sub, 128))
        s_pk = jnp.reshape(s, (sub, 128))
        mine = jnp.concatenate([m_pk, s_pk], axis=0)
        stats_ref[my] = mine

        pl.semaphore_wait(barrier_sem, N_DEV - 1)

        for k in range(N_DEV):
            @pl.when(k != my)
            def _():
                pltpu.make_async_remote_copy(
                    src_ref=stats_ref.at[my],
                    dst_ref=stats_ref.at[my],
                    send_sem=send_sems.at[k],
                    recv_sem=recv_sems.at[my],
                    device_id=(k,),
                    device_id_type=pl.DeviceIdType.MESH,
                ).start()

        out_ref[:, :] = p

        for k in range(N_DEV):
            @pl.when(k != my)
            def _():
                pltpu.make_async_remote_copy(
                    src_ref=stats_ref.at[k],
                    dst_ref=stats_ref.at[k],
                    send_sem=send_sems.at[k],
                    recv_sem=recv_sems.at[k],
                    device_id=(k,),
                    device_id_type=pl.DeviceIdType.MESH,
                ).wait_recv()

        g = stats_ref[:, :, :]
        gm = g[:, 0:sub, :]
        gs = g[:, sub:, :]
        gmax = jnp.max(gm, axis=0)
        denom = jnp.sum(gs * jnp.exp(gm - gmax), axis=0)
        scale_pk = jnp.exp(m_pk - gmax) / denom
        out_ref[:, :] = out_ref[:, :] * jnp.reshape(scale_pk, (m_per, 1))

        for k in range(N_DEV):
            @pl.when(k != my)
            def _():
                pltpu.make_async_remote_copy(
                    src_ref=stats_ref.at[my],
                    dst_ref=stats_ref.at[my],
                    send_sem=send_sems.at[k],
                    recv_sem=recv_sems.at[my],
                    device_id=(k,),
                    device_id_type=pl.DeviceIdType.MESH,
                ).wait_send()

    return pl.pallas_call(
        body,
        out_shape=jax.ShapeDtypeStruct((m_per, n_per), jnp.float32),
        in_specs=[pl.BlockSpec(memory_space=pltpu.VMEM)],
        out_specs=pl.BlockSpec(memory_space=pltpu.VMEM),
        scratch_shapes=[
            pltpu.VMEM((N_DEV, 8, 128), jnp.float32),
            pltpu.SemaphoreType.DMA((N_DEV,)),
            pltpu.SemaphoreType.DMA((N_DEV,)),
        ],
        compiler_params=pltpu.CompilerParams(collective_id=0),
    )(x)
